# Initial kernel scaffold; baseline (speedup 1.0000x reference)
import jax
import jax.numpy as jnp
from jax.experimental import pallas as pl


def kernel(xyzt, W1, b1, W2, b2, emb_0, emb_1, emb_2, emb_3, emb_4, emb_5, emb_6, emb_7, emb_8, emb_9, emb_10, emb_11, emb_12, emb_13, emb_14, emb_15):
    raise NotImplementedError("write your pallas kernel here")


# speedup vs baseline: 25.2310x; 25.2310x over previous
import functools

import jax
import jax.numpy as jnp
import numpy as np
from jax import lax
from jax.experimental import pallas as pl
from jax.experimental.pallas import tpu as pltpu
from jax.experimental.pallas import tpu_sc as plsc

N_LEVELS = 16
F_PER = 2
BASE = 16
FINEST = 64
N_PIECES = 10
B = 65536
OUT_DIM = N_LEVELS * F_PER
D_ROW = N_PIECES * F_PER
D_PAD = 32
XE_COLS = N_PIECES * OUT_DIM
_b = np.exp((np.log(FINEST) - np.log(BASE)) / (N_LEVELS - 1))
RES = [int(np.floor(BASE * _b ** i)) for i in range(N_LEVELS)]

NC = 2
NS = 16
NW = NC * NS
LANES = 16
PW = B // NW
PB = 128
NBLK = PW // PB
NG = PB // LANES


def _make_indices_weights(xyzt):
    xyz = xyzt[:, :3]
    idx_levels = []
    w_levels = []
    for res in RES:
        grid = (jnp.float32(1.0) - jnp.float32(0.0)) / res
        bot = jnp.floor(xyz / grid)
        vmin = bot * grid
        vmax = vmin + grid
        w = (xyz - vmin) / (vmax - vmin)
        b0 = jnp.clip(bot.astype(jnp.int32), 0, res - 1)
        b1c = jnp.minimum(b0 + 1, res - 1)
        cx = (b0[:, 0], b1c[:, 0])
        cy = (b0[:, 1], b1c[:, 1])
        cz = (b0[:, 2], b1c[:, 2])
        rows = []
        for ix in range(2):
            for iy in range(2):
                for iz in range(2):
                    rows.append((cx[ix] * res + cy[iy]) * res + cz[iz])
        idx_levels.append(jnp.stack(rows, axis=0))
        w_levels.append(jnp.transpose(w))
    return jnp.stack(idx_levels, axis=0), jnp.stack(w_levels, axis=0)


def _sc_gather_interp(idx_all, w_all, tabs):
    mesh = plsc.VectorSubcoreMesh(
        core_axis_name="c", subcore_axis_name="s", num_cores=NC, num_subcores=NS
    )

    @functools.partial(
        pl.kernel,
        mesh=mesh,
        compiler_params=pltpu.CompilerParams(
            needs_layout_passes=False, use_tc_tiling_on_sc=False
        ),
        out_type=jax.ShapeDtypeStruct((B, XE_COLS), jnp.float32),
        scratch_types=[
            pltpu.VMEM((8, PB), jnp.int32),
            pltpu.VMEM((3, PB), jnp.float32),
            pltpu.VMEM((8 * PB, D_PAD), jnp.float32),
            pltpu.VMEM((PB, XE_COLS), jnp.float32),
            pltpu.SemaphoreType.DMA,
        ],
    )
    def body(*refs):
        idx_hbm = refs[0]
        w_hbm = refs[1]
        tab_refs = refs[2:2 + N_LEVELS]
        xe_hbm = refs[2 + N_LEVELS]
        idx_v, wv, gbuf, out_s, sem = refs[3 + N_LEVELS:]

        wid = lax.axis_index("s") * NC + lax.axis_index("c")
        lane = lax.iota(jnp.int32, LANES)

        def blk_body(blk, carry):
            base = pl.multiple_of(wid * PW + blk * PB, PB)
            for L in range(N_LEVELS):
                pltpu.sync_copy(idx_hbm.at[L, :, pl.ds(base, PB)], idx_v)
                pltpu.sync_copy(w_hbm.at[L, :, pl.ds(base, PB)], wv)
                descs = [
                    pltpu.async_copy(
                        tab_refs[L].at[idx_v.at[c]],
                        gbuf.at[pl.ds(c * PB, PB)],
                        sem,
                    )
                    for c in range(8)
                ]
                for d in descs:
                    d.wait()

                def g_body(g, carry2):
                    goff = g * LANES
                    wx = wv[0, pl.ds(goff, LANES)]
                    wy = wv[1, pl.ds(goff, LANES)]
                    wz = wv[2, pl.ds(goff, LANES)]
                    ux = 1.0 - wx
                    uy = 1.0 - wy
                    uz = 1.0 - wz
                    p00 = ux * uy
                    p01 = ux * wy
                    p10 = wx * uy
                    p11 = wx * wy
                    s = (p00 * uz, p00 * wz, p01 * uz, p01 * wz,
                         p10 * uz, p10 * wz, p11 * uz, p11 * wz)
                    pvec = lane + goff
                    rowc = [pvec + c * PB for c in range(8)]

                    def j_body(j, carry3):
                        jv = jnp.full((LANES,), j, dtype=jnp.int32)
                        vals = [
                            plsc.load_gather(gbuf, [rowc[c], jv]) for c in range(8)
                        ]
                        acc = (
                            (s[0] * vals[0] + s[1] * vals[1])
                            + (s[2] * vals[2] + s[3] * vals[3])
                        ) + (
                            (s[4] * vals[4] + s[5] * vals[5])
                            + (s[6] * vals[6] + s[7] * vals[7])
                        )
                        col = (j // 2) * OUT_DIM + (2 * L + (j % 2))
                        colv = jnp.full((LANES,), col, dtype=jnp.int32)
                        plsc.store_scatter(out_s, [pvec, colv], acc)
                        return carry3

                    lax.fori_loop(0, D_ROW, j_body, 0)
                    return carry2

                lax.fori_loop(0, NG, g_body, 0)
            pltpu.sync_copy(out_s, xe_hbm.at[pl.ds(base, PB)])
            return carry

        lax.fori_loop(0, NBLK, blk_body, 0)

    return body(idx_all, w_all, *tabs)


def _tc_mlp(xe, xyzt, W1, b1, W2t, b2):
    R = 2048

    def body(xe_ref, xyzt_ref, W1_ref, b1_ref, W2t_ref, b2_ref, o_ref):
        x = xe_ref[...]
        t = xyzt_ref[:, 3:4]
        W1v = W1_ref[...]
        b1v = b1_ref[...]
        W2tv = W2t_ref[...]
        b2v = b2_ref[...]
        xs = []
        logits = []
        for p in range(N_PIECES):
            xp = x[:, p * OUT_DIM:(p + 1) * OUT_DIM]
            xs.append(xp)
            h = jnp.maximum(
                jnp.dot(xp, W1v, preferred_element_type=jnp.float32) + b1v, 0.0
            )
            a = jnp.sum(h * W2tv, axis=1, keepdims=True) + b2v
            an = 1.0 / (1.0 + jnp.exp(-a))
            logits.append(-jnp.abs(t - an) * jnp.float32(0.01))
        m = logits[0]
        for p in range(1, N_PIECES):
            m = jnp.maximum(m, logits[p])
        es = [jnp.exp(l - m) for l in logits]
        z = es[0]
        for p in range(1, N_PIECES):
            z = z + es[p]
        zinv = 1.0 / z
        acc = (es[0] * zinv) * xs[0]
        for p in range(1, N_PIECES):
            acc = acc + (es[p] * zinv) * xs[p]
        o_ref[...] = acc

    return pl.pallas_call(
        body,
        grid=(B // R,),
        in_specs=[
            pl.BlockSpec((R, XE_COLS), lambda i: (i, 0)),
            pl.BlockSpec((R, 4), lambda i: (i, 0)),
            pl.BlockSpec((OUT_DIM, 64), lambda i: (0, 0)),
            pl.BlockSpec((1, 64), lambda i: (0, 0)),
            pl.BlockSpec((1, 64), lambda i: (0, 0)),
            pl.BlockSpec((1, 1), lambda i: (0, 0)),
        ],
        out_specs=pl.BlockSpec((R, OUT_DIM), lambda i: (i, 0)),
        out_shape=jax.ShapeDtypeStruct((B, OUT_DIM), jnp.float32),
    )(xe, xyzt, W1, b1, W2t, b2)


def kernel(xyzt, W1, b1, W2, b2, emb_0, emb_1, emb_2, emb_3, emb_4, emb_5,
           emb_6, emb_7, emb_8, emb_9, emb_10, emb_11, emb_12, emb_13,
           emb_14, emb_15):
    embs = [emb_0, emb_1, emb_2, emb_3, emb_4, emb_5, emb_6, emb_7, emb_8,
            emb_9, emb_10, emb_11, emb_12, emb_13, emb_14, emb_15]
    tabs = [
        jnp.pad(e.reshape(RES[i] ** 3, D_ROW), ((0, 0), (0, D_PAD - D_ROW)))
        for i, e in enumerate(embs)
    ]
    idx_all, w_all = _make_indices_weights(xyzt)
    xe = _sc_gather_interp(idx_all, w_all, tabs)
    return _tc_mlp(xe, xyzt, W1, b1.reshape(1, 64), W2.reshape(1, 64),
                   b2.reshape(1, 1))


# speedup vs baseline: 26.2177x; 1.0391x over previous
import functools

import jax
import jax.numpy as jnp
import numpy as np
from jax import lax
from jax.experimental import pallas as pl
from jax.experimental.pallas import tpu as pltpu
from jax.experimental.pallas import tpu_sc as plsc

N_LEVELS = 16
F_PER = 2
BASE = 16
FINEST = 64
N_PIECES = 10
B = 65536
OUT_DIM = N_LEVELS * F_PER
D_ROW = N_PIECES * F_PER
XE_COLS = N_PIECES * OUT_DIM
_b = np.exp((np.log(FINEST) - np.log(BASE)) / (N_LEVELS - 1))
RES = [int(np.floor(BASE * _b ** i)) for i in range(N_LEVELS)]

NC = 2
NS = 16
NW = NC * NS
LANES = 16
PW = B // NW
PB = 128
NBLK = PW // PB
NG = PB // LANES


def _make_indices_weights(xyzt):
    xyz = xyzt[:, :3]
    idx_levels = []
    w_levels = []
    for res in RES:
        grid = (jnp.float32(1.0) - jnp.float32(0.0)) / res
        bot = jnp.floor(xyz / grid)
        vmin = bot * grid
        vmax = vmin + grid
        w = (xyz - vmin) / (vmax - vmin)
        b0 = jnp.clip(bot.astype(jnp.int32), 0, res - 1)
        b1c = jnp.minimum(b0 + 1, res - 1)
        cx = (b0[:, 0], b1c[:, 0])
        cy = (b0[:, 1], b1c[:, 1])
        cz = (b0[:, 2], b1c[:, 2])
        rows = []
        for ix in range(2):
            for iy in range(2):
                for iz in range(2):
                    v = (cx[ix] * res + cy[iy]) * res + cz[iz]
                    f0 = 20 * v
                    r32 = f0 >> 5
                    off = f0 & 31
                    rows.append((r32 << 5) | off)
        idx_levels.append(jnp.stack(rows, axis=0))
        w_levels.append(jnp.transpose(w))
    return jnp.stack(idx_levels, axis=0), jnp.stack(w_levels, axis=0)


def _sc_gather_interp(idx_all, w_all, tabs):
    mesh = plsc.VectorSubcoreMesh(
        core_axis_name="c", subcore_axis_name="s", num_cores=NC, num_subcores=NS
    )

    @functools.partial(
        pl.kernel,
        mesh=mesh,
        compiler_params=pltpu.CompilerParams(
            needs_layout_passes=False, use_tc_tiling_on_sc=False
        ),
        out_type=(
            jax.ShapeDtypeStruct((B, 128), jnp.float32),
            jax.ShapeDtypeStruct((B, 128), jnp.float32),
            jax.ShapeDtypeStruct((B, 128), jnp.float32),
        ),
        scratch_types=[
            pltpu.VMEM((8, PB), jnp.int32),
            pltpu.VMEM((8, PB), jnp.int32),
            pltpu.VMEM((8, PB), jnp.int32),
            pltpu.VMEM((3, PB), jnp.float32),
            pltpu.VMEM((8 * PB, 32), jnp.float32),
            pltpu.VMEM((8 * PB, 32), jnp.float32),
            pltpu.VMEM((PB, XE_COLS), jnp.float32),
            pltpu.SemaphoreType.DMA,
        ],
    )
    def body(*refs):
        idx_hbm = refs[0]
        w_hbm = refs[1]
        tab_refs = refs[2:2 + N_LEVELS]
        xa_hbm, xb_hbm, xc_hbm = refs[2 + N_LEVELS:5 + N_LEVELS]
        pidx_v, r0_v, r1_v, wv, g0, g1, out_s, sem = refs[5 + N_LEVELS:]

        wid = lax.axis_index("s") * NC + lax.axis_index("c")
        lane = lax.iota(jnp.int32, LANES)

        def blk_body(blk, carry):
            base = pl.multiple_of(wid * PW + blk * PB, PB)
            for L in range(N_LEVELS):
                pltpu.sync_copy(idx_hbm.at[L, :, pl.ds(base, PB)], pidx_v)
                pltpu.sync_copy(w_hbm.at[L, :, pl.ds(base, PB)], wv)

                def unpack_body(g, carry2):
                    goff = g * LANES
                    for c in range(8):
                        pv = pidx_v[c, pl.ds(goff, LANES)]
                        r0 = lax.shift_right_logical(pv, 5)
                        r0_v[c, pl.ds(goff, LANES)] = r0
                        r1_v[c, pl.ds(goff, LANES)] = r0 + 1
                    return carry2

                lax.fori_loop(0, NG, unpack_body, 0)
                descs = []
                for c in range(8):
                    descs.append(pltpu.async_copy(
                        tab_refs[L].at[r0_v.at[c]],
                        g0.at[pl.ds(c * PB, PB)], sem))
                    descs.append(pltpu.async_copy(
                        tab_refs[L].at[r1_v.at[c]],
                        g1.at[pl.ds(c * PB, PB)], sem))
                for d in descs:
                    d.wait()

                def g_body(g, carry2):
                    goff = g * LANES
                    wx = wv[0, pl.ds(goff, LANES)]
                    wy = wv[1, pl.ds(goff, LANES)]
                    wz = wv[2, pl.ds(goff, LANES)]
                    ux = 1.0 - wx
                    uy = 1.0 - wy
                    uz = 1.0 - wz
                    p00 = ux * uy
                    p01 = ux * wy
                    p10 = wx * uy
                    p11 = wx * wy
                    s = (p00 * uz, p00 * wz, p01 * uz, p01 * wz,
                         p10 * uz, p10 * wz, p11 * uz, p11 * wz)
                    pvec = lane + goff
                    rowc = [pvec + c * PB for c in range(8)]
                    offc = [pidx_v[c, pl.ds(goff, LANES)] & 31 for c in range(8)]

                    def j_body(j, carry3):
                        cols = [offc[c] + j for c in range(8)]
                        vals = []
                        for c in range(8):
                            col = cols[c]
                            v0 = plsc.load_gather(
                                g0, [rowc[c], jnp.minimum(col, 31)])
                            v1 = plsc.load_gather(
                                g1, [rowc[c], jnp.maximum(col - 32, 0)])
                            vals.append(jnp.where(col < 32, v0, v1))
                        acc = (
                            (s[0] * vals[0] + s[1] * vals[1])
                            + (s[2] * vals[2] + s[3] * vals[3])
                        ) + (
                            (s[4] * vals[4] + s[5] * vals[5])
                            + (s[6] * vals[6] + s[7] * vals[7])
                        )
                        col = (j // 2) * OUT_DIM + (2 * L + (j % 2))
                        colv = jnp.full((LANES,), col, dtype=jnp.int32)
                        plsc.store_scatter(out_s, [pvec, colv], acc)
                        return carry3

                    lax.fori_loop(0, D_ROW, j_body, 0)
                    return carry2

                lax.fori_loop(0, NG, g_body, 0)
            pltpu.sync_copy(out_s.at[:, pl.ds(0, 128)],
                            xa_hbm.at[pl.ds(base, PB)])
            pltpu.sync_copy(out_s.at[:, pl.ds(128, 128)],
                            xb_hbm.at[pl.ds(base, PB)])
            pltpu.sync_copy(out_s.at[:, pl.ds(256, 64)],
                            xc_hbm.at[pl.ds(base, PB), pl.ds(0, 64)])
            return carry

        lax.fori_loop(0, NBLK, blk_body, 0)

    return body(idx_all, w_all, *tabs)


def _tc_mlp(xa, xb, xc, xyzt, W1, b1, W2t, b2):
    R = 2048

    def body(xa_ref, xb_ref, xc_ref, xyzt_ref, W1_ref, b1_ref, W2t_ref,
             b2_ref, o_ref):
        x = jnp.concatenate(
            [xa_ref[...], xb_ref[...], xc_ref[:, :64]], axis=-1)
        t = xyzt_ref[:, 3:4]
        W1v = W1_ref[...]
        b1v = b1_ref[...]
        W2tv = W2t_ref[...]
        b2v = b2_ref[...]
        xs = []
        logits = []
        for p in range(N_PIECES):
            xp = x[:, p * OUT_DIM:(p + 1) * OUT_DIM]
            xs.append(xp)
            h = jnp.maximum(
                jnp.dot(xp, W1v, preferred_element_type=jnp.float32) + b1v, 0.0
            )
            a = jnp.sum(h * W2tv, axis=1, keepdims=True) + b2v
            an = 1.0 / (1.0 + jnp.exp(-a))
            logits.append(-jnp.abs(t - an) * jnp.float32(0.01))
        m = logits[0]
        for p in range(1, N_PIECES):
            m = jnp.maximum(m, logits[p])
        es = [jnp.exp(l - m) for l in logits]
        z = es[0]
        for p in range(1, N_PIECES):
            z = z + es[p]
        zinv = 1.0 / z
        acc = (es[0] * zinv) * xs[0]
        for p in range(1, N_PIECES):
            acc = acc + (es[p] * zinv) * xs[p]
        o_ref[...] = acc

    return pl.pallas_call(
        body,
        grid=(B // R,),
        in_specs=[
            pl.BlockSpec((R, 128), lambda i: (i, 0)),
            pl.BlockSpec((R, 128), lambda i: (i, 0)),
            pl.BlockSpec((R, 128), lambda i: (i, 0)),
            pl.BlockSpec((R, 4), lambda i: (i, 0)),
            pl.BlockSpec((OUT_DIM, 64), lambda i: (0, 0)),
            pl.BlockSpec((1, 64), lambda i: (0, 0)),
            pl.BlockSpec((1, 64), lambda i: (0, 0)),
            pl.BlockSpec((1, 1), lambda i: (0, 0)),
        ],
        out_specs=pl.BlockSpec((R, OUT_DIM), lambda i: (i, 0)),
        out_shape=jax.ShapeDtypeStruct((B, OUT_DIM), jnp.float32),
    )(xa, xb, xc, xyzt, W1, b1, W2t, b2)


def kernel(xyzt, W1, b1, W2, b2, emb_0, emb_1, emb_2, emb_3, emb_4, emb_5,
           emb_6, emb_7, emb_8, emb_9, emb_10, emb_11, emb_12, emb_13,
           emb_14, emb_15):
    embs = [emb_0, emb_1, emb_2, emb_3, emb_4, emb_5, emb_6, emb_7, emb_8,
            emb_9, emb_10, emb_11, emb_12, emb_13, emb_14, emb_15]
    tabs = []
    for i, e in enumerate(embs):
        flat = e.reshape(-1)
        pad = (-flat.shape[0]) % 32 + 32
        flat = jnp.pad(flat, (0, pad))
        tabs.append(flat.reshape(-1, 32))
    idx_all, w_all = _make_indices_weights(xyzt)
    xa, xb, xc = _sc_gather_interp(idx_all, w_all, tabs)
    return _tc_mlp(xa, xb, xc, xyzt, W1, b1.reshape(1, 64), W2.reshape(1, 64),
                   b2.reshape(1, 1))


# speedup vs baseline: 26.2253x; 1.0003x over previous
import functools

import jax
import jax.numpy as jnp
import numpy as np
from jax import lax
from jax.experimental import pallas as pl
from jax.experimental.pallas import tpu as pltpu
from jax.experimental.pallas import tpu_sc as plsc

N_LEVELS = 16
F_PER = 2
BASE = 16
FINEST = 64
N_PIECES = 10
B = 65536
OUT_DIM = N_LEVELS * F_PER
D_ROW = N_PIECES * F_PER
XE_COLS = N_PIECES * OUT_DIM
_b = np.exp((np.log(FINEST) - np.log(BASE)) / (N_LEVELS - 1))
RES = [int(np.floor(BASE * _b ** i)) for i in range(N_LEVELS)]

NC = 2
NS = 16
NW = NC * NS
LANES = 16
PW = B // NW
PB = 128
NBLK = PW // PB
NG = PB // LANES


def _make_indices_weights(xyzt):
    xyz = xyzt[:, :3]
    idx_levels = []
    w_levels = []
    for res in RES:
        grid = (jnp.float32(1.0) - jnp.float32(0.0)) / res
        bot = jnp.floor(xyz / grid)
        vmin = bot * grid
        vmax = vmin + grid
        w = (xyz - vmin) / (vmax - vmin)
        b0 = jnp.clip(bot.astype(jnp.int32), 0, res - 1)
        b1c = jnp.minimum(b0 + 1, res - 1)
        cx = (b0[:, 0], b1c[:, 0])
        cy = (b0[:, 1], b1c[:, 1])
        cz = (b0[:, 2], b1c[:, 2])
        rows = []
        for ix in range(2):
            for iy in range(2):
                for iz in range(2):
                    v = (cx[ix] * res + cy[iy]) * res + cz[iz]
                    f0 = 20 * v
                    r32 = f0 >> 5
                    off = f0 & 31
                    rows.append((r32 << 5) | off)
        idx_levels.append(jnp.stack(rows, axis=0))
        w_levels.append(jnp.transpose(w))
    return jnp.stack(idx_levels, axis=0), jnp.stack(w_levels, axis=0)


def _sc_gather_interp(idx_all, w_all, tabs):
    tabs_nrows = [t.shape[0] for t in tabs]
    mesh = plsc.VectorSubcoreMesh(
        core_axis_name="c", subcore_axis_name="s", num_cores=NC, num_subcores=NS
    )

    @functools.partial(
        pl.kernel,
        mesh=mesh,
        compiler_params=pltpu.CompilerParams(
            needs_layout_passes=False, use_tc_tiling_on_sc=False
        ),
        out_type=(
            jax.ShapeDtypeStruct((B, 128), jnp.float32),
            jax.ShapeDtypeStruct((B, 128), jnp.float32),
            jax.ShapeDtypeStruct((B, 128), jnp.float32),
        ),
        scratch_types=[
            pltpu.VMEM((8, PB), jnp.int32),
            pltpu.VMEM((8, PB), jnp.int32),
            pltpu.VMEM((8, PB), jnp.int32),
            pltpu.VMEM((3, PB), jnp.float32),
            pltpu.VMEM((8 * PB, 32), jnp.float32),
            pltpu.VMEM((8 * PB, 32), jnp.float32),
            pltpu.VMEM((PB, XE_COLS), jnp.float32),
            pltpu.SemaphoreType.DMA,
        ],
    )
    def body(*refs):
        idx_hbm = refs[0]
        w_hbm = refs[1]
        tab_refs = refs[2:2 + N_LEVELS]
        xa_hbm, xb_hbm, xc_hbm = refs[2 + N_LEVELS:5 + N_LEVELS]
        pidx_v, r0_v, r1_v, wv, g0, g1, out_s, sem = refs[5 + N_LEVELS:]

        wid = lax.axis_index("s") * NC + lax.axis_index("c")
        lane = lax.iota(jnp.int32, LANES)

        def blk_body(blk, carry):
            base = pl.multiple_of(wid * PW + blk * PB, PB)
            for L in range(N_LEVELS):
                pltpu.sync_copy(idx_hbm.at[L, :, pl.ds(base, PB)], pidx_v)
                pltpu.sync_copy(w_hbm.at[L, :, pl.ds(base, PB)], wv)

                rmax = tabs_nrows[L] - 1

                def unpack_body(g, carry2):
                    goff = g * LANES
                    for c in range(8):
                        pv = pidx_v[c, pl.ds(goff, LANES)]
                        r0 = lax.shift_right_logical(pv, 5)
                        r0_v[c, pl.ds(goff, LANES)] = r0
                        r1_v[c, pl.ds(goff, LANES)] = jnp.minimum(r0 + 1, rmax)
                    return carry2

                lax.fori_loop(0, NG, unpack_body, 0)
                descs = []
                for c in range(8):
                    descs.append(pltpu.async_copy(
                        tab_refs[L].at[r0_v.at[c]],
                        g0.at[pl.ds(c * PB, PB)], sem))
                    descs.append(pltpu.async_copy(
                        tab_refs[L].at[r1_v.at[c]],
                        g1.at[pl.ds(c * PB, PB)], sem))
                for d in descs:
                    d.wait()

                def g_body(g, carry2):
                    goff = g * LANES
                    wx = wv[0, pl.ds(goff, LANES)]
                    wy = wv[1, pl.ds(goff, LANES)]
                    wz = wv[2, pl.ds(goff, LANES)]
                    ux = 1.0 - wx
                    uy = 1.0 - wy
                    uz = 1.0 - wz
                    p00 = ux * uy
                    p01 = ux * wy
                    p10 = wx * uy
                    p11 = wx * wy
                    s = (p00 * uz, p00 * wz, p01 * uz, p01 * wz,
                         p10 * uz, p10 * wz, p11 * uz, p11 * wz)
                    pvec = lane + goff
                    rowc = [pvec + c * PB for c in range(8)]
                    offc = [pidx_v[c, pl.ds(goff, LANES)] & 31 for c in range(8)]

                    def j_body(j, carry3):
                        cols = [offc[c] + j for c in range(8)]
                        vals = []
                        for c in range(8):
                            col = cols[c]
                            v0 = plsc.load_gather(
                                g0, [rowc[c], jnp.minimum(col, 31)])
                            v1 = plsc.load_gather(
                                g1, [rowc[c], jnp.maximum(col - 32, 0)])
                            vals.append(jnp.where(col < 32, v0, v1))
                        acc = (
                            (s[0] * vals[0] + s[1] * vals[1])
                            + (s[2] * vals[2] + s[3] * vals[3])
                        ) + (
                            (s[4] * vals[4] + s[5] * vals[5])
                            + (s[6] * vals[6] + s[7] * vals[7])
                        )
                        col = (j // 2) * OUT_DIM + (2 * L + (j % 2))
                        colv = jnp.full((LANES,), col, dtype=jnp.int32)
                        plsc.store_scatter(out_s, [pvec, colv], acc)
                        return carry3

                    lax.fori_loop(0, D_ROW, j_body, 0)
                    return carry2

                lax.fori_loop(0, NG, g_body, 0)
            pltpu.sync_copy(out_s.at[:, pl.ds(0, 128)],
                            xa_hbm.at[pl.ds(base, PB)])
            pltpu.sync_copy(out_s.at[:, pl.ds(128, 128)],
                            xb_hbm.at[pl.ds(base, PB)])
            pltpu.sync_copy(out_s.at[:, pl.ds(256, 64)],
                            xc_hbm.at[pl.ds(base, PB), pl.ds(0, 64)])
            return carry

        lax.fori_loop(0, NBLK, blk_body, 0)

    return body(idx_all, w_all, *tabs)


def _tc_mlp(xa, xb, xc, xyzt, W1, b1, W2t, b2):
    R = 2048

    def body(xa_ref, xb_ref, xc_ref, xyzt_ref, W1_ref, b1_ref, W2t_ref,
             b2_ref, o_ref):
        x = jnp.concatenate(
            [xa_ref[...], xb_ref[...], xc_ref[:, :64]], axis=-1)
        t = xyzt_ref[:, 3:4]
        W1v = W1_ref[...]
        b1v = b1_ref[...]
        W2tv = W2t_ref[...]
        b2v = b2_ref[...]
        xs = []
        logits = []
        for p in range(N_PIECES):
            xp = x[:, p * OUT_DIM:(p + 1) * OUT_DIM]
            xs.append(xp)
            h = jnp.maximum(
                jnp.dot(xp, W1v, preferred_element_type=jnp.float32) + b1v, 0.0
            )
            a = jnp.sum(h * W2tv, axis=1, keepdims=True) + b2v
            an = 1.0 / (1.0 + jnp.exp(-a))
            logits.append(-jnp.abs(t - an) * jnp.float32(0.01))
        m = logits[0]
        for p in range(1, N_PIECES):
            m = jnp.maximum(m, logits[p])
        es = [jnp.exp(l - m) for l in logits]
        z = es[0]
        for p in range(1, N_PIECES):
            z = z + es[p]
        zinv = 1.0 / z
        acc = (es[0] * zinv) * xs[0]
        for p in range(1, N_PIECES):
            acc = acc + (es[p] * zinv) * xs[p]
        o_ref[...] = acc

    return pl.pallas_call(
        body,
        grid=(B // R,),
        in_specs=[
            pl.BlockSpec((R, 128), lambda i: (i, 0)),
            pl.BlockSpec((R, 128), lambda i: (i, 0)),
            pl.BlockSpec((R, 128), lambda i: (i, 0)),
            pl.BlockSpec((R, 4), lambda i: (i, 0)),
            pl.BlockSpec((OUT_DIM, 64), lambda i: (0, 0)),
            pl.BlockSpec((1, 64), lambda i: (0, 0)),
            pl.BlockSpec((1, 64), lambda i: (0, 0)),
            pl.BlockSpec((1, 1), lambda i: (0, 0)),
        ],
        out_specs=pl.BlockSpec((R, OUT_DIM), lambda i: (i, 0)),
        out_shape=jax.ShapeDtypeStruct((B, OUT_DIM), jnp.float32),
    )(xa, xb, xc, xyzt, W1, b1, W2t, b2)


def kernel(xyzt, W1, b1, W2, b2, emb_0, emb_1, emb_2, emb_3, emb_4, emb_5,
           emb_6, emb_7, emb_8, emb_9, emb_10, emb_11, emb_12, emb_13,
           emb_14, emb_15):
    embs = [emb_0, emb_1, emb_2, emb_3, emb_4, emb_5, emb_6, emb_7, emb_8,
            emb_9, emb_10, emb_11, emb_12, emb_13, emb_14, emb_15]
    tabs = []
    for i, e in enumerate(embs):
        flat = e.reshape(-1)
        if flat.shape[0] % 32 == 0:
            tabs.append(flat.reshape(-1, 32))
        else:
            pad = (-flat.shape[0]) % 32 + 32
            flat = jnp.pad(flat, (0, pad))
            tabs.append(flat.reshape(-1, 32))
    idx_all, w_all = _make_indices_weights(xyzt)
    xa, xb, xc = _sc_gather_interp(idx_all, w_all, tabs)
    return _tc_mlp(xa, xb, xc, xyzt, W1, b1.reshape(1, 64), W2.reshape(1, 64),
                   b2.reshape(1, 1))
